# baseline (speedup 1.0000x reference)
import jax
import jax.numpy as jnp
from jax import lax
from jax.experimental import pallas as pl
from jax.experimental.pallas import tpu as pltpu
from jax.experimental.pallas import tpu_sc as plsc

_ROWS = 32
_COLS = 32768
_N = _ROWS * _COLS
_K = 512 * _ROWS
_NSUB = 16
_C = _N // _NSUB
_NV = _C // 16
_CAP = 16400
_CCAP = 16384


def _tokey(v):
    s = lax.shift_right_logical(lax.shift_right_arithmetic(v, 31), 1)
    return v ^ s


def _suffix_search(ref, nvec, kneed, base):
    def outer_cond(carry):
        jj, _, jf, _ = carry
        return (jj < nvec) & (jf < 0)

    def outer(carry):
        jj, acc, jf, accb = carry
        j = nvec - 1 - jj
        s = jnp.sum(ref[pl.ds(base + 16 * j, 16)])
        newacc = acc + s
        crossed = newacc >= kneed
        jf = jnp.where(crossed, j, jf)
        accb = jnp.where(crossed, acc, accb)
        return jj + 1, newacc, jf, accb

    _, _, jf, accb = lax.while_loop(
        outer_cond,
        outer,
        (jnp.int32(0), jnp.int32(0), jnp.int32(-1), jnp.int32(0)),
    )

    hv = ref[pl.ds(base + 16 * jf, 16)]
    cum = plsc.cumsum(hv)
    total = cum[15]
    suffix_incl = accb + total - cum + hv
    ntrue = jnp.sum((suffix_incl >= kneed).astype(jnp.int32))
    bl = ntrue - 1
    lanes = lax.iota(jnp.int32, 16)
    cum_at = jnp.sum(jnp.where(lanes == bl, cum, 0))
    bf = 16 * jf + bl
    mf = accb + total - cum_at
    return bf, mf


def _sc_body(
    x_hbm,
    out_hbm,
    keys,
    hist,
    mhist,
    tmp,
    h3,
    cbufk,
    cbufi,
    stage,
    smem,
    sem,
    sem4,
    sh_hist,
    sh_merged,
):
    sid = lax.axis_index("s")
    base = sid * _C
    ones = jnp.ones((16,), jnp.int32)
    zeros = jnp.zeros((16,), jnp.int32)
    lanes = lax.iota(jnp.int32, 16)

    in_handles = [
        pltpu.async_copy(
            x_hbm.at[pl.ds(base + q * (_C // 4), _C // 4)],
            keys.at[pl.ds(q * (_C // 4), _C // 4)],
            sem4.at[q],
        )
        for q in range(4)
    ]

    def zero_hist():
        @plsc.parallel_loop(0, 256, unroll=4)
        def _zero(_i):
            hist[pl.ds(16 * _i, 16)] = zeros

    zero_hist()

    for q in range(4):
        in_handles[q].wait()

        @plsc.parallel_loop(q * (_NV // 4), (q + 1) * (_NV // 4), unroll=4)
        def pass_a(i):
            v = _tokey(keys[pl.ds(16 * i, 16)])
            keys[pl.ds(16 * i, 16)] = v
            b = lax.shift_right_arithmetic(v, 20) + 2048
            plsc.addupdate_scatter(hist, [b], ones)

    def merge_4096():
        pltpu.sync_copy(hist, sh_hist.at[sid])
        plsc.subcore_barrier()
        col = sid * 256

        def merge_row(r, c):
            pltpu.sync_copy(sh_hist.at[r, pl.ds(col, 256)], tmp)

            def addv(vv, cc):
                sl = pl.ds(16 * vv, 16)
                mhist[sl] = jnp.where(r == 0, tmp[sl], mhist[sl] + tmp[sl])
                return cc

            lax.fori_loop(0, 16, addv, 0)
            return c

        lax.fori_loop(0, 16, merge_row, 0)
        pltpu.sync_copy(mhist.at[pl.ds(0, 256)], sh_merged.at[pl.ds(col, 256)])
        plsc.subcore_barrier()
        pltpu.sync_copy(sh_merged, mhist)

    merge_4096()
    b1, m1 = _suffix_search(mhist, 256, _K, 0)

    zero_hist()
    hi1 = b1 - 2048

    @plsc.parallel_loop(0, _NV, unroll=2, carry=jnp.int32(0))
    def pass_b(i, c):
        v = keys[pl.ds(16 * i, 16)]
        msk = lax.shift_right_arithmetic(v, 20) == hi1
        b = lax.shift_right_arithmetic(v, 8) & 0xFFF
        plsc.addupdate_scatter(hist, [b], ones, mask=msk)
        off = jnp.minimum(c, _CCAP - 16)
        smsk = msk & (c <= _CCAP - 16)
        plsc.store_compressed(cbufk.at[pl.ds(off, 16)], v, mask=smsk)
        plsc.store_compressed(cbufi.at[pl.ds(off, 16)], 16 * i + lanes, mask=smsk)
        return c + jnp.sum(msk.astype(jnp.int32))

    n_cand = pass_b
    ovf = n_cand > _CCAP - 16
    nv_cand0 = (jnp.minimum(n_cand, _CCAP) + 15) // 16
    merge_4096()
    kneed2 = _K - m1
    b2, m2p = _suffix_search(mhist, 256, kneed2, 0)
    m2 = m1 + m2p
    p24 = hi1 * 4096 + b2

    zero_hist()
    nv_cand = nv_cand0

    @plsc.parallel_loop(0, nv_cand, unroll=2)
    def pass_c(j):
        v = cbufk[pl.ds(16 * j, 16)]
        valid = (16 * j + lanes) < n_cand
        msk = valid & (lax.shift_right_arithmetic(v, 8) == p24)
        b = v & 0xFF
        plsc.addupdate_scatter(hist, [b], ones, mask=msk)

    @pl.when(ovf)
    def _pass_c_full():
        @plsc.parallel_loop(0, 256, unroll=4)
        def _zero(_i):
            hist[pl.ds(16 * _i, 16)] = zeros

        @plsc.parallel_loop(0, _NV, unroll=4)
        def _full(i):
            v = keys[pl.ds(16 * i, 16)]
            msk = lax.shift_right_arithmetic(v, 8) == p24
            plsc.addupdate_scatter(hist, [v & 0xFF], ones, mask=msk)

    pltpu.sync_copy(hist.at[pl.ds(0, 256)], sh_hist.at[sid, pl.ds(0, 256)])
    plsc.subcore_barrier()

    def fetch_row(r, c):
        pltpu.sync_copy(sh_hist.at[r, pl.ds(0, 256)], mhist.at[pl.ds(256 * r, 256)])
        return c

    lax.fori_loop(0, 16, fetch_row, 0)

    def rowsum(vv, c):
        def addr(r, acc):
            return acc + mhist[pl.ds(256 * r + 16 * vv, 16)]

        h3[pl.ds(16 * vv, 16)] = lax.fori_loop(0, 16, addr, zeros)
        return c

    lax.fori_loop(0, 16, rowsum, 0)

    kneed3 = _K - m2
    b3, m3p = _suffix_search(h3, 16, kneed3, 0)
    m3 = m2 + m3p
    t_key = p24 * 256 + b3
    r_quota = _K - m3

    tr = plsc.load_gather(mhist, [256 * lanes + b3])
    p_w = jnp.sum(jnp.where(lanes < sid, tr, 0))
    t_w = jnp.sum(jnp.where(lanes == sid, tr, 0))
    q_w = jnp.clip(r_quota - p_w, 0, t_w)

    def d1c(j, carry):
        rank, ck = carry
        kv = cbufk[pl.ds(16 * j, 16)]
        pos = cbufi[pl.ds(16 * j, 16)]
        valid = (16 * j + lanes) < n_cand
        tie = valid & (kv == t_key)
        tc = tie.astype(jnp.int32)
        rank_v = rank + plsc.cumsum(tc)
        keep = tie & (rank_v <= q_w)
        plsc.store_compressed(cbufi.at[pl.ds(ck, 16)], pos, mask=keep)
        return rank + jnp.sum(tc), ck + jnp.sum(keep.astype(jnp.int32))

    _, ckept_fast = lax.fori_loop(
        0, nv_cand, d1c, (jnp.int32(0), jnp.int32(0))
    )

    @pl.when(ovf)
    def _d1_full():
        def d1_cond(carry):
            i, rank, ck = carry
            return (i < _NV) & (ck < q_w)

        def d1_body(carry):
            i, rank, ck = carry
            v = keys[pl.ds(16 * i, 16)]
            tie = v == t_key
            tc = tie.astype(jnp.int32)
            rank_v = rank + plsc.cumsum(tc)
            keep = tie & (rank_v <= q_w)
            pos = 16 * i + lanes
            plsc.store_compressed(cbufi.at[pl.ds(ck, 16)], pos, mask=keep)
            ck = ck + jnp.sum(keep.astype(jnp.int32))
            rank = rank + jnp.sum(tc)
            return i + 1, rank, ck

        _, _, ck = lax.while_loop(
            d1_cond, d1_body, (jnp.int32(0), jnp.int32(0), jnp.int32(0))
        )
        smem[0] = ck

    ckept = jnp.where(ovf, smem[0], ckept_fast)

    @plsc.parallel_loop(0, _NV, unroll=4)
    def pass_d2(i):
        v = keys[pl.ds(16 * i, 16)]
        keys[pl.ds(16 * i, 16)] = jnp.where(v > t_key, _tokey(v), 0)

    inv_t = t_key ^ lax.shift_right_logical(
        lax.shift_right_arithmetic(t_key, 31), 1
    )
    tie_val = jnp.full((16,), inv_t, jnp.int32)

    def pass_d3(j, c):
        posv = cbufi[pl.ds(16 * j, 16)]
        msk = (16 * j + lanes) < ckept
        plsc.store_scatter(keys, [posv], tie_val, mask=msk)
        return c

    lax.fori_loop(0, (ckept + 15) // 16, pass_d3, 0)

    pltpu.sync_copy(keys, out_hbm.at[pl.ds(base, _C)])


def kernel(x):
    bits = jax.lax.bitcast_convert_type(x, jnp.int32).reshape(_N)
    mesh = plsc.VectorSubcoreMesh(
        core_axis_name="c", subcore_axis_name="s", num_cores=1, num_subcores=_NSUB
    )
    out_bits = pl.kernel(
        _sc_body,
        out_type=jax.ShapeDtypeStruct((_N,), jnp.int32),
        mesh=mesh,
        compiler_params=pltpu.CompilerParams(
            use_tc_tiling_on_sc=False, needs_layout_passes=False
        ),
        scratch_types=[
            pltpu.VMEM((_C,), jnp.int32),
            pltpu.VMEM((4096,), jnp.int32),
            pltpu.VMEM((4096,), jnp.int32),
            pltpu.VMEM((256,), jnp.int32),
            pltpu.VMEM((256,), jnp.int32),
            pltpu.VMEM((_CCAP,), jnp.int32),
            pltpu.VMEM((_CAP,), jnp.int32),
            pltpu.VMEM((4096,), jnp.int32),
            pltpu.SMEM((8,), jnp.int32),
            pltpu.SemaphoreType.DMA,
            pltpu.SemaphoreType.DMA((4,)),
            pltpu.VMEM_SHARED((16, 4096), jnp.int32),
            pltpu.VMEM_SHARED((4096,), jnp.int32),
        ],
    )(bits)
    return jax.lax.bitcast_convert_type(out_bits.reshape(_ROWS, _COLS), jnp.float32)


# speedup vs baseline: 1.1613x; 1.1613x over previous
import jax
import jax.numpy as jnp
from jax import lax
from jax.experimental import pallas as pl
from jax.experimental.pallas import tpu as pltpu
from jax.experimental.pallas import tpu_sc as plsc

_ROWS = 32
_COLS = 32768
_N = _ROWS * _COLS
_K = 512 * _ROWS
_NSUB = 16
_C = _N // _NSUB
_NV = _C // 16
_CAP = 16400
_CCAP = 16384


def _tokey(v):
    s = lax.shift_right_logical(lax.shift_right_arithmetic(v, 31), 1)
    return v ^ s


def _suffix_search(ref, nvec, kneed, base):
    def outer_cond(carry):
        jj, _, jf, _ = carry
        return (jj < nvec) & (jf < 0)

    def outer(carry):
        jj, acc, jf, accb = carry
        j = nvec - 1 - jj
        s = jnp.sum(ref[pl.ds(base + 16 * j, 16)])
        newacc = acc + s
        crossed = newacc >= kneed
        jf = jnp.where(crossed, j, jf)
        accb = jnp.where(crossed, acc, accb)
        return jj + 1, newacc, jf, accb

    _, _, jf, accb = lax.while_loop(
        outer_cond,
        outer,
        (jnp.int32(0), jnp.int32(0), jnp.int32(-1), jnp.int32(0)),
    )

    hv = ref[pl.ds(base + 16 * jf, 16)]
    cum = plsc.cumsum(hv)
    total = cum[15]
    suffix_incl = accb + total - cum + hv
    ntrue = jnp.sum((suffix_incl >= kneed).astype(jnp.int32))
    bl = ntrue - 1
    lanes = lax.iota(jnp.int32, 16)
    cum_at = jnp.sum(jnp.where(lanes == bl, cum, 0))
    bf = 16 * jf + bl
    mf = accb + total - cum_at
    return bf, mf


def _sc_body(
    x_hbm,
    out_hbm,
    keys,
    hist,
    mhist,
    tmp,
    h3,
    cbufk,
    cbufi,
    stage,
    smem,
    sem,
    sem4,
    sh_hist,
    sh_merged,
):
    sid = lax.axis_index("s")
    base = sid * _C
    ones = jnp.ones((16,), jnp.int32)
    zeros = jnp.zeros((16,), jnp.int32)
    lanes = lax.iota(jnp.int32, 16)

    in_handles = [
        pltpu.async_copy(
            x_hbm.at[pl.ds(base + q * (_C // 4), _C // 4)],
            keys.at[pl.ds(q * (_C // 4), _C // 4)],
            sem4.at[q],
        )
        for q in range(4)
    ]

    def zero_hist():
        @plsc.parallel_loop(0, 256, unroll=4)
        def _zero(_i):
            hist[pl.ds(16 * _i, 16)] = zeros

    zero_hist()

    for q in range(4):
        in_handles[q].wait()

        @plsc.parallel_loop(q * (_NV // 4), (q + 1) * (_NV // 4), unroll=4)
        def pass_a(i):
            v = _tokey(keys[pl.ds(16 * i, 16)])
            keys[pl.ds(16 * i, 16)] = v
            b = lax.shift_right_arithmetic(v, 20) + 2048
            plsc.addupdate_scatter(hist, [b], ones)

    def merge_4096():
        pltpu.sync_copy(hist, sh_hist.at[sid])
        plsc.subcore_barrier()
        col = sid * 256

        def merge_row(r, c):
            pltpu.sync_copy(sh_hist.at[r, pl.ds(col, 256)], tmp)

            def addv(vv, cc):
                sl = pl.ds(16 * vv, 16)
                mhist[sl] = jnp.where(r == 0, tmp[sl], mhist[sl] + tmp[sl])
                return cc

            lax.fori_loop(0, 16, addv, 0)
            return c

        lax.fori_loop(0, 16, merge_row, 0)
        pltpu.sync_copy(mhist.at[pl.ds(0, 256)], sh_merged.at[pl.ds(col, 256)])
        plsc.subcore_barrier()
        pltpu.sync_copy(sh_merged, mhist)

    merge_4096()
    b1, m1 = _suffix_search(mhist, 256, _K, 0)

    zero_hist()
    hi1 = b1 - 2048

    @plsc.parallel_loop(0, _NV, unroll=4, carry=jnp.int32(0))
    def pass_b(i, c):
        v = keys[pl.ds(16 * i, 16)]
        msk = lax.shift_right_arithmetic(v, 20) == hi1
        b = lax.shift_right_arithmetic(v, 8) & 0xFFF
        plsc.addupdate_scatter(hist, [b], ones, mask=msk)
        off = jnp.minimum(c, _CCAP - 16)
        smsk = msk & (c <= _CCAP - 16)
        plsc.store_compressed(cbufk.at[pl.ds(off, 16)], v, mask=smsk)
        plsc.store_compressed(cbufi.at[pl.ds(off, 16)], 16 * i + lanes, mask=smsk)
        return c + jnp.sum(msk.astype(jnp.int32))

    n_cand = pass_b
    ovf = n_cand > _CCAP - 16
    nv_cand0 = (jnp.minimum(n_cand, _CCAP) + 15) // 16
    merge_4096()
    kneed2 = _K - m1
    b2, m2p = _suffix_search(mhist, 256, kneed2, 0)
    m2 = m1 + m2p
    p24 = hi1 * 4096 + b2

    zero_hist()
    nv_cand = nv_cand0

    @plsc.parallel_loop(0, nv_cand, unroll=2)
    def pass_c(j):
        v = cbufk[pl.ds(16 * j, 16)]
        valid = (16 * j + lanes) < n_cand
        msk = valid & (lax.shift_right_arithmetic(v, 8) == p24)
        b = v & 0xFF
        plsc.addupdate_scatter(hist, [b], ones, mask=msk)

    @pl.when(ovf)
    def _pass_c_full():
        @plsc.parallel_loop(0, 256, unroll=4)
        def _zero(_i):
            hist[pl.ds(16 * _i, 16)] = zeros

        @plsc.parallel_loop(0, _NV, unroll=4)
        def _full(i):
            v = keys[pl.ds(16 * i, 16)]
            msk = lax.shift_right_arithmetic(v, 8) == p24
            plsc.addupdate_scatter(hist, [v & 0xFF], ones, mask=msk)

    pltpu.sync_copy(hist.at[pl.ds(0, 256)], sh_hist.at[sid, pl.ds(0, 256)])
    plsc.subcore_barrier()

    def fetch_row(r, c):
        pltpu.sync_copy(sh_hist.at[r, pl.ds(0, 256)], mhist.at[pl.ds(256 * r, 256)])
        return c

    lax.fori_loop(0, 16, fetch_row, 0)

    def rowsum(vv, c):
        def addr(r, acc):
            return acc + mhist[pl.ds(256 * r + 16 * vv, 16)]

        h3[pl.ds(16 * vv, 16)] = lax.fori_loop(0, 16, addr, zeros)
        return c

    lax.fori_loop(0, 16, rowsum, 0)

    kneed3 = _K - m2
    b3, m3p = _suffix_search(h3, 16, kneed3, 0)
    m3 = m2 + m3p
    t_key = p24 * 256 + b3
    r_quota = _K - m3

    tr = plsc.load_gather(mhist, [256 * lanes + b3])
    p_w = jnp.sum(jnp.where(lanes < sid, tr, 0))
    t_w = jnp.sum(jnp.where(lanes == sid, tr, 0))
    q_w = jnp.clip(r_quota - p_w, 0, t_w)

    def d1c(j, carry):
        rank, ck = carry
        kv = cbufk[pl.ds(16 * j, 16)]
        pos = cbufi[pl.ds(16 * j, 16)]
        valid = (16 * j + lanes) < n_cand
        tie = valid & (kv == t_key)
        tc = tie.astype(jnp.int32)
        rank_v = rank + plsc.cumsum(tc)
        keep = tie & (rank_v <= q_w)
        plsc.store_compressed(cbufi.at[pl.ds(ck, 16)], pos, mask=keep)
        return rank + jnp.sum(tc), ck + jnp.sum(keep.astype(jnp.int32))

    _, ckept_fast = lax.fori_loop(
        0, nv_cand, d1c, (jnp.int32(0), jnp.int32(0))
    )

    @pl.when(ovf)
    def _d1_full():
        def d1_cond(carry):
            i, rank, ck = carry
            return (i < _NV) & (ck < q_w)

        def d1_body(carry):
            i, rank, ck = carry
            v = keys[pl.ds(16 * i, 16)]
            tie = v == t_key
            tc = tie.astype(jnp.int32)
            rank_v = rank + plsc.cumsum(tc)
            keep = tie & (rank_v <= q_w)
            pos = 16 * i + lanes
            plsc.store_compressed(cbufi.at[pl.ds(ck, 16)], pos, mask=keep)
            ck = ck + jnp.sum(keep.astype(jnp.int32))
            rank = rank + jnp.sum(tc)
            return i + 1, rank, ck

        _, _, ck = lax.while_loop(
            d1_cond, d1_body, (jnp.int32(0), jnp.int32(0), jnp.int32(0))
        )
        smem[0] = ck

    ckept = jnp.where(ovf, smem[0], ckept_fast)

    inv_t = t_key ^ lax.shift_right_logical(
        lax.shift_right_arithmetic(t_key, 31), 1
    )
    tie_val = jnp.full((16,), inv_t, jnp.int32)
    nv_tie = (ckept + 15) // 16
    out_handles = []
    for q in range(4):
        @plsc.parallel_loop(q * (_NV // 4), (q + 1) * (_NV // 4), unroll=4)
        def pass_d2(i):
            v = keys[pl.ds(16 * i, 16)]
            keys[pl.ds(16 * i, 16)] = jnp.where(v > t_key, _tokey(v), 0)

        def pass_d3(j, c):
            posv = cbufi[pl.ds(16 * j, 16)]
            msk = (16 * j + lanes) < ckept
            inq = (posv >= q * (_C // 4)) & (posv < (q + 1) * (_C // 4))
            plsc.store_scatter(keys, [posv], tie_val, mask=msk & inq)
            return c

        lax.fori_loop(0, nv_tie, pass_d3, 0)
        out_handles.append(
            pltpu.async_copy(
                keys.at[pl.ds(q * (_C // 4), _C // 4)],
                out_hbm.at[pl.ds(base + q * (_C // 4), _C // 4)],
                sem4.at[q],
            )
        )

    for h in out_handles:
        h.wait()


def kernel(x):
    bits = jax.lax.bitcast_convert_type(x, jnp.int32).reshape(_N)
    mesh = plsc.VectorSubcoreMesh(
        core_axis_name="c", subcore_axis_name="s", num_cores=1, num_subcores=_NSUB
    )
    out_bits = pl.kernel(
        _sc_body,
        out_type=jax.ShapeDtypeStruct((_N,), jnp.int32),
        mesh=mesh,
        compiler_params=pltpu.CompilerParams(
            use_tc_tiling_on_sc=False, needs_layout_passes=False
        ),
        scratch_types=[
            pltpu.VMEM((_C,), jnp.int32),
            pltpu.VMEM((4096,), jnp.int32),
            pltpu.VMEM((4096,), jnp.int32),
            pltpu.VMEM((256,), jnp.int32),
            pltpu.VMEM((256,), jnp.int32),
            pltpu.VMEM((_CCAP,), jnp.int32),
            pltpu.VMEM((_CAP,), jnp.int32),
            pltpu.VMEM((4096,), jnp.int32),
            pltpu.SMEM((8,), jnp.int32),
            pltpu.SemaphoreType.DMA,
            pltpu.SemaphoreType.DMA((4,)),
            pltpu.VMEM_SHARED((16, 4096), jnp.int32),
            pltpu.VMEM_SHARED((4096,), jnp.int32),
        ],
    )(bits)
    return jax.lax.bitcast_convert_type(out_bits.reshape(_ROWS, _COLS), jnp.float32)
